# baseline (speedup 1.0000x reference)
import jax
import jax.numpy as jnp
from jax.experimental import pallas as pl
from jax.experimental.pallas import tpu as pltpu

_N = 100000
_C = 3
_W = 40
_PR = _N // _W
_PC = _C * _W


def _mask_body(idx_ref, out_ref):
    out_ref[...] = jnp.zeros(out_ref.shape, jnp.bool_)
    lane = jax.lax.broadcasted_iota(jnp.int32, (1, _PC), 1)
    for k in range(idx_ref.shape[0]):
        i = idx_ref[k]
        r = i // _W
        c = _C * jax.lax.rem(i, _W)
        m = (lane >= c) & (lane < c + _C)
        row = out_ref[pl.ds(r, 1), :]
        out_ref[pl.ds(r, 1), :] = jnp.logical_or(row, m)


def kernel(pos, idx):
    del pos
    idx32 = idx.astype(jnp.int32)
    packed = pl.pallas_call(
        _mask_body,
        out_shape=jax.ShapeDtypeStruct((_PR, _PC), jnp.bool_),
        in_specs=[pl.BlockSpec(memory_space=pltpu.SMEM)],
    )(idx32)
    return packed.reshape(_N, _C)


# speedup vs baseline: 10.0847x; 10.0847x over previous
import jax
import jax.numpy as jnp
from jax.experimental import pallas as pl
from jax.experimental.pallas import tpu as pltpu

_N = 100000
_C = 3
_L = 128
_NP = ((_N + _L - 1) // _L) * _L
_R = 4


def _mask_body(idx_ref, out_ref):
    out_ref[...] = jnp.zeros(out_ref.shape, jnp.bool_)
    lane = jax.lax.broadcasted_iota(jnp.int32, (_R, _L), 1)
    for k in range(idx_ref.shape[0]):
        i = idx_ref[k]
        t = (i // _L) * _L
        onehot = lane == (i - t)
        out_ref[:, pl.ds(t, _L)] = jnp.logical_or(out_ref[:, pl.ds(t, _L)],
                                                  onehot)


def kernel(pos, idx):
    del pos
    idx32 = idx.astype(jnp.int32)
    packed = pl.pallas_call(
        _mask_body,
        out_shape=jax.ShapeDtypeStruct((_R, _NP), jnp.bool_),
        in_specs=[pl.BlockSpec(memory_space=pltpu.SMEM)],
    )(idx32)
    return packed[:_C, :_N].T


# speedup vs baseline: 12.5526x; 1.2447x over previous
import jax
import jax.numpy as jnp
from jax.experimental import pallas as pl
from jax.experimental.pallas import tpu as pltpu

_N = 100000
_C = 3
_L = 128
_NP = ((_N + _L - 1) // _L) * _L
_T = _NP // _L


def _mask_body(idx_ref, out_ref):
    out_ref[...] = jnp.zeros(out_ref.shape, jnp.int32)
    lane = jax.lax.broadcasted_iota(jnp.int32, (1, _L), 1)
    for k in range(idx_ref.shape[0]):
        i = idx_ref[k]
        t = i // _L
        onehot = jnp.where(lane == (i - t * _L), jnp.int32(1), jnp.int32(0))
        row = out_ref[pl.ds(t, 1), :]
        out_ref[pl.ds(t, 1), :] = row | onehot


def kernel(pos, idx):
    del pos
    idx32 = idx.astype(jnp.int32)
    words = pl.pallas_call(
        _mask_body,
        out_shape=jax.ShapeDtypeStruct((_T, _L), jnp.int32),
        in_specs=[pl.BlockSpec(memory_space=pltpu.SMEM)],
    )(idx32)
    return words.reshape(_NP)[:_N, None] != jnp.zeros((1, _C), jnp.int32)
